# baseline (speedup 1.0000x reference)
import jax
import jax.numpy as jnp
from jax import lax
from jax.experimental import pallas as pl
from jax.experimental.pallas import tpu as pltpu
from jax.experimental.pallas import tpu_sc as plsc

VOCAB = 1000000
HIDDEN = 64
BATCH = 16384
SEQ = 16

_B = BATCH * SEQ
_NC, _NS = 2, 16
_NW = _NC * _NS
_B_PER_W = _B // _NW
_CHUNK = 128
_N_CHUNKS = _B_PER_W // _CHUNK


def _gather_body(ids_hbm, table_hbm, out_hbm, idx_v, rows_v, sem):
    wid = lax.axis_index("s") * _NC + lax.axis_index("c")
    pltpu.sync_copy(ids_hbm.at[pl.ds(wid * _N_CHUNKS, _N_CHUNKS)], idx_v)

    def step(j, carry):
        pltpu.async_copy(table_hbm.at[idx_v.at[j]], rows_v, sem).wait()
        pltpu.sync_copy(
            rows_v, out_hbm.at[pl.ds(wid * _B_PER_W + j * _CHUNK, _CHUNK)]
        )
        return carry

    lax.fori_loop(0, _N_CHUNKS, step, 0)


_gather = pl.kernel(
    _gather_body,
    mesh=plsc.VectorSubcoreMesh(core_axis_name="c", subcore_axis_name="s"),
    out_type=jax.ShapeDtypeStruct((_B, HIDDEN), jnp.float32),
    scratch_types=[
        pltpu.VMEM((_N_CHUNKS, _CHUNK), jnp.int32),
        pltpu.VMEM((_CHUNK, HIDDEN), jnp.float32),
        pltpu.SemaphoreType.DMA,
    ],
    compiler_params=pltpu.CompilerParams(use_tc_tiling_on_sc=False),
)


def _mask_body(am_ref, out_ref):
    out_ref[...] = am_ref[...] == 0


def kernel(input_ids, attention_mask, table):
    ids = input_ids.reshape(_NW * _N_CHUNKS, _CHUNK).astype(jnp.int32)
    emb = _gather(ids, table)
    mask = pl.pallas_call(
        _mask_body,
        out_shape=jax.ShapeDtypeStruct((BATCH // 128, 128 * SEQ), jnp.bool_),
    )(attention_mask.reshape(BATCH // 128, 128 * SEQ))
    return emb.reshape(BATCH, SEQ, HIDDEN), mask.reshape(BATCH, SEQ)
